# baseline (speedup 1.0000x reference)
import functools

import jax
import jax.numpy as jnp
from jax import lax
from jax.experimental import pallas as pl
from jax.experimental.pallas import tpu as pltpu
from jax.experimental.pallas import tpu_sc as plsc

N = 10000
E = 160000
NC, NS = 2, 16
NW = NC * NS
CHUNK = 64
KINF = 4
EPAD = 163840
CH2 = EPAD // (NS * CHUNK)
CHW = EPAD // (NW * CHUNK)
NACC = 10016
STRIPE = NACC // NS
RB = 400
GRID = N // RB

CALLS = {128: [("col", 64, 2)], 256: [("col", 64, 4)], 16: [("edge", 16)]}


def _parts_of(h):
    out = []
    for c in CALLS[h]:
        if c[0] == "col":
            out += [c[1]] * c[2]
        else:
            out.append(c[1])
    return out


PARTS = {h: _parts_of(h) for h in CALLS}


def _sc_mesh():
    return plsc.VectorSubcoreMesh(core_axis_name="c", subcore_axis_name="s")


def _sc_propagate_call(u_parts, src_g, dst_s, zeros, w, nparts):
    @functools.partial(
        pl.kernel,
        out_type=[jax.ShapeDtypeStruct((NACC, w), jnp.float32)] * nparts,
        mesh=_sc_mesh(),
        compiler_params=pltpu.CompilerParams(use_tc_tiling_on_sc=False),
        scratch_types=[
            pltpu.VMEM((CH2, CHUNK), jnp.int32),
            pltpu.VMEM((CH2, CHUNK), jnp.int32),
            pltpu.VMEM((KINF, CHUNK, w), jnp.float32),
            pltpu.VMEM_SHARED((N, w), jnp.float32),
            pltpu.VMEM_SHARED((NACC, w), jnp.float32),
            pltpu.SemaphoreType.DMA,
            pltpu.SemaphoreType.DMA,
        ],
    )
    def k(*refs):
        u_hbms = refs[:nparts]
        srcg_hbm, dsts_hbm, zeros_hbm = refs[nparts:nparts + 3]
        outs = refs[nparts + 3:2 * nparts + 3]
        src_v, dst_v, rows_v, u_sp, acc, sem_g, sem_s = refs[2 * nparts + 3:]
        c = lax.axis_index("c")
        s = lax.axis_index("s")

        def sc_body(part):
            @pl.when(s == 0)
            def _():
                pltpu.sync_copy(u_hbms[part], u_sp)

            pltpu.sync_copy(srcg_hbm.at[s], src_v)
            pltpu.sync_copy(dsts_hbm.at[s], dst_v)
            pltpu.sync_copy(zeros_hbm, acc.at[pl.ds(s * STRIPE, STRIPE)])
            plsc.subcore_barrier()

            def block(j, carry):
                base = j * KINF
                gets = [pltpu.async_copy(u_sp.at[src_v.at[base + t]],
                                         rows_v.at[t], sem_g)
                        for t in range(KINF)]
                puts = []
                for t in range(KINF):
                    gets[t].wait()
                    puts.append(pltpu.async_copy(rows_v.at[t],
                                                 acc.at[dst_v.at[base + t]],
                                                 sem_s, add=True))
                for d in puts:
                    d.wait()
                return carry

            lax.fori_loop(0, CH2 // KINF, block, 0)
            plsc.subcore_barrier()
            pltpu.sync_copy(acc.at[pl.ds(s * STRIPE, STRIPE)],
                            outs[part].at[pl.ds(s * STRIPE, STRIPE)])

        @pl.when(c == 0)
        def _():
            sc_body(0)
            if nparts == 4:
                sc_body(2)

        @pl.when(c == 1)
        def _():
            sc_body(1)
            if nparts == 4:
                sc_body(3)

    res = k(*u_parts, src_g, dst_s, zeros)
    return list(res) if isinstance(res, (list, tuple)) else [res]


def _sc_propagate_edge(u, src_gw, dst_sw, zeros, w):
    @functools.partial(
        pl.kernel,
        out_type=[jax.ShapeDtypeStruct((NACC, w), jnp.float32)] * 2,
        mesh=_sc_mesh(),
        compiler_params=pltpu.CompilerParams(use_tc_tiling_on_sc=False),
        scratch_types=[
            pltpu.VMEM((CHW, CHUNK), jnp.int32),
            pltpu.VMEM((CHW, CHUNK), jnp.int32),
            pltpu.VMEM((KINF, CHUNK, w), jnp.float32),
            pltpu.VMEM_SHARED((N, w), jnp.float32),
            pltpu.VMEM_SHARED((NACC, w), jnp.float32),
            pltpu.SemaphoreType.DMA,
            pltpu.SemaphoreType.DMA,
        ],
    )
    def k(u_hbm, srcg_hbm, dsts_hbm, zeros_hbm, out0, out1,
          src_v, dst_v, rows_v, u_sp, acc, sem_g, sem_s):
        c = lax.axis_index("c")
        s = lax.axis_index("s")
        widx = c * NS + s

        @pl.when(s == 0)
        def _():
            pltpu.sync_copy(u_hbm, u_sp)

        pltpu.sync_copy(srcg_hbm.at[widx], src_v)
        pltpu.sync_copy(dsts_hbm.at[widx], dst_v)
        pltpu.sync_copy(zeros_hbm, acc.at[pl.ds(s * STRIPE, STRIPE)])
        plsc.subcore_barrier()

        def block(j, carry):
            base = j * KINF
            gets = [pltpu.async_copy(u_sp.at[src_v.at[base + t]],
                                     rows_v.at[t], sem_g)
                    for t in range(KINF)]
            puts = []
            for t in range(KINF):
                gets[t].wait()
                puts.append(pltpu.async_copy(rows_v.at[t],
                                             acc.at[dst_v.at[base + t]],
                                             sem_s, add=True))
            for d in puts:
                d.wait()
            return carry

        lax.fori_loop(0, CHW // KINF, block, 0)
        plsc.subcore_barrier()

        @pl.when(c == 0)
        def _():
            pltpu.sync_copy(acc.at[pl.ds(s * STRIPE, STRIPE)],
                            out0.at[pl.ds(s * STRIPE, STRIPE)])

        @pl.when(c == 1)
        def _():
            pltpu.sync_copy(acc.at[pl.ds(s * STRIPE, STRIPE)],
                            out1.at[pl.ds(s * STRIPE, STRIPE)])

    return list(k(u, src_gw, dst_sw, zeros))


def _sc_propagate(u_parts, edges, zeros, h):
    (src_g, dst_s), (src_gw, dst_sw) = edges
    outs, i = [], 0
    for call in CALLS[h]:
        if call[0] == "col":
            _, w, n = call
            outs += _sc_propagate_call(u_parts[i:i + n], src_g, dst_s,
                                       zeros[w], w, n)
            i += n
        else:
            w = call[1]
            outs += _sc_propagate_edge(u_parts[i], src_gw, dst_sw,
                                       zeros[w], w)
            i += 1
    return outs


def _sc_degree(src_s, ones, zeros):
    @functools.partial(
        pl.kernel,
        out_type=jax.ShapeDtypeStruct((NC, NACC, 16), jnp.float32),
        mesh=_sc_mesh(),
        compiler_params=pltpu.CompilerParams(use_tc_tiling_on_sc=False),
        scratch_types=[
            pltpu.VMEM((CHW, CHUNK), jnp.int32),
            pltpu.VMEM((CHUNK, 16), jnp.float32),
            pltpu.VMEM_SHARED((NACC, 16), jnp.float32),
            pltpu.SemaphoreType.DMA,
        ],
    )
    def k(srcs_hbm, ones_hbm, zeros_hbm, out_hbm, src_v, ones_v, acc, sem_s):
        c = lax.axis_index("c")
        s = lax.axis_index("s")
        widx = c * NS + s
        pltpu.sync_copy(srcs_hbm.at[widx], src_v)
        pltpu.sync_copy(ones_hbm, ones_v)
        pltpu.sync_copy(zeros_hbm, acc.at[pl.ds(s * STRIPE, STRIPE)])
        plsc.subcore_barrier()

        def block(j, carry):
            base = j * KINF
            puts = [pltpu.async_copy(ones_v, acc.at[src_v.at[base + t]],
                                     sem_s, add=True)
                    for t in range(KINF)]
            for d in puts:
                d.wait()
            return carry

        lax.fori_loop(0, CHW // KINF, block, 0)
        plsc.subcore_barrier()
        pltpu.sync_copy(acc.at[pl.ds(s * STRIPE, STRIPE)],
                        out_hbm.at[c, pl.ds(s * STRIPE, STRIPE)])

    return k(src_s, ones, zeros)


def _dis_tc(deg_p):
    def body(p0_ref, p1_ref, o_ref):
        d = p0_ref[0] + p1_ref[0]
        o_ref[...] = jnp.where(d > 0, lax.rsqrt(d), 0.0)

    return pl.pallas_call(
        body,
        grid=(GRID,),
        in_specs=[pl.BlockSpec((1, RB, 16), lambda i: (0, i, 0)),
                  pl.BlockSpec((1, RB, 16), lambda i: (1, i, 0))],
        out_specs=pl.BlockSpec((RB, 16), lambda i: (i, 0)),
        out_shape=jax.ShapeDtypeStruct((N, 16), jnp.float32),
    )(deg_p, deg_p)


def _col_offsets(h):
    offs, o = [], 0
    for w in PARTS[h]:
        offs.append(o)
        o += w
    return offs


def _prescale(x, dis, h):
    parts = PARTS[h]
    offs = _col_offsets(h)

    def body(x_ref, d_ref, *o_refs):
        u = x_ref[...] * d_ref[:, 0:1]
        for r, w, o in zip(o_refs, parts, offs):
            r[...] = u[:, o:o + w]

    return pl.pallas_call(
        body,
        grid=(GRID,),
        in_specs=[pl.BlockSpec((RB, h), lambda i: (i, 0)),
                  pl.BlockSpec((RB, 16), lambda i: (i, 0))],
        out_specs=[pl.BlockSpec((RB, w), lambda i: (i, 0)) for w in parts],
        out_shape=[jax.ShapeDtypeStruct((N, w), jnp.float32) for w in parts],
    )(x, dis)


def _combine(p_parts, dis, a, terms, h, relu=False, bias=None, want_u=True):
    parts = PARTS[h]
    offs = _col_offsets(h)
    np_ = len(p_parts)
    g = np_ // len(parts)
    nt = len(terms)
    nb = 1 if bias is not None else 0

    def body(*refs):
        ps = [sum(refs[i * g + j][...] for j in range(g))
              for i in range(len(parts))]
        psum = ps[0] if len(parts) == 1 else jnp.concatenate(ps, axis=1)
        dcol = refs[np_][:, 0:1]
        t = a * dcol * psum
        for (_, sgn), r in zip(terms, refs[np_ + 1:np_ + 1 + nt]):
            t = t + sgn * r[...]
        if bias is not None:
            t = t + refs[np_ + 1 + nt][...]
        if relu:
            t = jnp.maximum(t, 0.0)
        out0 = np_ + 1 + nt + nb
        refs[out0][...] = t
        if want_u:
            u = dcol * t
            for i, (w, o) in enumerate(zip(parts, offs)):
                refs[out0 + 1 + i][...] = u[:, o:o + w]

    in_specs, args = [], []
    pws = [w for w in parts for _ in range(g)]
    for p, w in zip(p_parts, pws):
        in_specs.append(pl.BlockSpec((RB, w), lambda i: (i, 0)))
        args.append(p)
    in_specs.append(pl.BlockSpec((RB, 16), lambda i: (i, 0)))
    args.append(dis)
    for (arr, _) in terms:
        in_specs.append(pl.BlockSpec((RB, h), lambda i: (i, 0)))
        args.append(arr)
    if bias is not None:
        in_specs.append(pl.BlockSpec((1, h), lambda i: (0, 0)))
        args.append(bias)
    out_shape = [jax.ShapeDtypeStruct((N, h), jnp.float32)]
    out_specs = [pl.BlockSpec((RB, h), lambda i: (i, 0))]
    if want_u:
        for w in parts:
            out_shape.append(jax.ShapeDtypeStruct((N, w), jnp.float32))
            out_specs.append(pl.BlockSpec((RB, w), lambda i: (i, 0)))
    res = pl.pallas_call(
        body, grid=(GRID,), in_specs=in_specs,
        out_specs=out_specs, out_shape=out_shape,
    )(*args)
    if want_u:
        return res[0], list(res[1:])
    return res[0]


def _mm_cheb4(ts, w, b):
    f, c = w.shape[1], w.shape[2]

    def body(t0, t1, t2, t3, w_ref, b_ref, o_ref):
        acc = jnp.dot(t0[...], w_ref[0], preferred_element_type=jnp.float32)
        acc = acc + jnp.dot(t1[...], w_ref[1], preferred_element_type=jnp.float32)
        acc = acc + jnp.dot(t2[...], w_ref[2], preferred_element_type=jnp.float32)
        acc = acc + jnp.dot(t3[...], w_ref[3], preferred_element_type=jnp.float32)
        o_ref[...] = jnp.maximum(acc + b_ref[...], 0.0)

    return pl.pallas_call(
        body,
        grid=(GRID,),
        in_specs=[pl.BlockSpec((RB, f), lambda i: (i, 0)),
                  pl.BlockSpec((RB, f), lambda i: (i, 0)),
                  pl.BlockSpec((RB, f), lambda i: (i, 0)),
                  pl.BlockSpec((RB, f), lambda i: (i, 0)),
                  pl.BlockSpec((4, f, c), lambda i: (0, 0, 0)),
                  pl.BlockSpec((1, c), lambda i: (0, 0))],
        out_specs=pl.BlockSpec((RB, c), lambda i: (i, 0)),
        out_shape=jax.ShapeDtypeStruct((N, c), jnp.float32),
    )(ts[0], ts[1], ts[2], ts[3], w, b)


def _mm_split(hmat, wp):
    f, c = wp.shape
    h = c // 4

    def body(h_ref, w_ref, o0, o1, o2, o3):
        acc = jnp.dot(h_ref[...], w_ref[...],
                      preferred_element_type=jnp.float32)
        o0[...] = acc[:, 0 * h:1 * h]
        o1[...] = acc[:, 1 * h:2 * h]
        o2[...] = acc[:, 2 * h:3 * h]
        o3[...] = acc[:, 3 * h:4 * h]

    return pl.pallas_call(
        body,
        grid=(GRID,),
        in_specs=[pl.BlockSpec((RB, f), lambda i: (i, 0)),
                  pl.BlockSpec((f, c), lambda i: (0, 0))],
        out_specs=[pl.BlockSpec((RB, h), lambda i: (i, 0))] * 4,
        out_shape=[jax.ShapeDtypeStruct((N, h), jnp.float32)] * 4,
    )(hmat, wp)


def _layer1(x, dis, edges, zeros, w1, b1):
    u0 = _prescale(x, dis, 128)
    p1 = _sc_propagate(u0, edges, zeros, 128)
    t1, u1 = _combine(p1, dis, -1.0, [], 128)
    p2 = _sc_propagate(u1, edges, zeros, 128)
    t2, u2 = _combine(p2, dis, -2.0, [(x, -1.0)], 128)
    p3 = _sc_propagate(u2, edges, zeros, 128)
    t3 = _combine(p3, dis, -2.0, [(t1, -1.0)], 128, want_u=False)
    return _mm_cheb4([x, t1, t2, t3], w1, b1)


def _layer_clenshaw(zs, dis, edges, zeros, h, bias):
    u3 = _prescale(zs[3], dis, h)
    p = _sc_propagate(u3, edges, zeros, h)
    c2, u2 = _combine(p, dis, -2.0, [(zs[2], 1.0)], h)
    p = _sc_propagate(u2, edges, zeros, h)
    c1, u1 = _combine(p, dis, -2.0, [(zs[1], 1.0), (zs[3], -1.0)], h)
    p = _sc_propagate(u1, edges, zeros, h)
    out = _combine(p, dis, -1.0, [(zs[0], 1.0), (c2, -1.0)], h,
                   relu=True, bias=bias, want_u=False)
    return out


def kernel(x, edge_index, W1, b1, W2, b2, W3, b3):
    ei = edge_index.astype(jnp.int32)
    src, dst = ei[0], ei[1]
    pad = EPAD - E
    src_g = jnp.pad(src, (0, pad)).reshape(NS, CH2, CHUNK)
    dst_s = jnp.pad(dst, (0, pad), constant_values=N).reshape(NS, CH2, CHUNK)
    src_gw = jnp.pad(src, (0, pad)).reshape(NW, CHW, CHUNK)
    dst_sw = jnp.pad(dst, (0, pad), constant_values=N).reshape(NW, CHW, CHUNK)
    src_s = jnp.pad(src, (0, pad), constant_values=N).reshape(NW, CHW, CHUNK)
    edges = ((src_g, dst_s), (src_gw, dst_sw))

    ones16 = jnp.ones((CHUNK, 16), jnp.float32)
    zeros = {w: jnp.zeros((STRIPE, w), jnp.float32) for w in (64, 16)}

    w2p = jnp.pad(W2, ((0, 0), (0, 0), (0, 56))).transpose(1, 0, 2).reshape(400, 4 * 256)
    w3p = jnp.pad(W3, ((0, 0), (0, 56), (0, 12))).transpose(1, 0, 2).reshape(256, 4 * 16)
    b1r = b1.reshape(1, 400)
    b2p = jnp.pad(b2, (0, 56)).reshape(1, 256)
    b3p = jnp.pad(b3, (0, 12)).reshape(1, 16)

    deg_p = _sc_degree(src_s, ones16, zeros[16])
    dis = _dis_tc(deg_p)

    h1 = _layer1(x, dis, edges, zeros, W1, b1r)
    z2 = _mm_split(h1, w2p)
    h2 = _layer_clenshaw(z2, dis, edges, zeros, 256, b2p)
    z3 = _mm_split(h2, w3p)
    h3 = _layer_clenshaw(z3, dis, edges, zeros, 16, b3p)
    return h3[:, :4]


# speedup vs baseline: 1.0415x; 1.0415x over previous
import functools

import jax
import jax.numpy as jnp
from jax import lax
from jax.experimental import pallas as pl
from jax.experimental.pallas import tpu as pltpu
from jax.experimental.pallas import tpu_sc as plsc

N = 10000
E = 160000
NC, NS = 2, 16
NW = NC * NS
CHUNK = 64
KINF = 4
EPAD = 163840
CH2 = EPAD // (NS * CHUNK)
CHW = EPAD // (NW * CHUNK)
NACC = 10016
STRIPE = NACC // NS
RB = 400
GRID = N // RB

CALLS = {128: [("col", 64, 2)],
         224: [("col", 64, 2), ("col", 48, 2)],
         16: [("edge", 16)]}


def _parts_of(h):
    out = []
    for c in CALLS[h]:
        if c[0] == "col":
            out += [c[1]] * c[2]
        else:
            out.append(c[1])
    return out


PARTS = {h: _parts_of(h) for h in CALLS}


def _sc_mesh():
    return plsc.VectorSubcoreMesh(core_axis_name="c", subcore_axis_name="s")


def _sc_propagate_call(u_parts, src_g, dst_s, zeros, w, nparts):
    @functools.partial(
        pl.kernel,
        out_type=[jax.ShapeDtypeStruct((NACC, w), jnp.float32)] * nparts,
        mesh=_sc_mesh(),
        compiler_params=pltpu.CompilerParams(use_tc_tiling_on_sc=False),
        scratch_types=[
            pltpu.VMEM((CH2, CHUNK), jnp.int32),
            pltpu.VMEM((CH2, CHUNK), jnp.int32),
            pltpu.VMEM((KINF, CHUNK, w), jnp.float32),
            pltpu.VMEM_SHARED((N, w), jnp.float32),
            pltpu.VMEM_SHARED((NACC, w), jnp.float32),
            pltpu.SemaphoreType.DMA,
            pltpu.SemaphoreType.DMA,
        ],
    )
    def k(*refs):
        u_hbms = refs[:nparts]
        srcg_hbm, dsts_hbm, zeros_hbm = refs[nparts:nparts + 3]
        outs = refs[nparts + 3:2 * nparts + 3]
        src_v, dst_v, rows_v, u_sp, acc, sem_g, sem_s = refs[2 * nparts + 3:]
        c = lax.axis_index("c")
        s = lax.axis_index("s")

        def sc_body(part):
            @pl.when(s == 0)
            def _():
                pltpu.sync_copy(u_hbms[part], u_sp)

            pltpu.sync_copy(srcg_hbm.at[s], src_v)
            pltpu.sync_copy(dsts_hbm.at[s], dst_v)
            pltpu.sync_copy(zeros_hbm, acc.at[pl.ds(s * STRIPE, STRIPE)])
            plsc.subcore_barrier()

            def block(j, carry):
                base = j * KINF
                gets = [pltpu.async_copy(u_sp.at[src_v.at[base + t]],
                                         rows_v.at[t], sem_g)
                        for t in range(KINF)]
                puts = []
                for t in range(KINF):
                    gets[t].wait()
                    puts.append(pltpu.async_copy(rows_v.at[t],
                                                 acc.at[dst_v.at[base + t]],
                                                 sem_s, add=True))
                for d in puts:
                    d.wait()
                return carry

            lax.fori_loop(0, CH2 // KINF, block, 0)
            plsc.subcore_barrier()
            pltpu.sync_copy(acc.at[pl.ds(s * STRIPE, STRIPE)],
                            outs[part].at[pl.ds(s * STRIPE, STRIPE)])

        @pl.when(c == 0)
        def _():
            sc_body(0)
            if nparts == 4:
                sc_body(2)

        @pl.when(c == 1)
        def _():
            sc_body(1)
            if nparts == 4:
                sc_body(3)

    res = k(*u_parts, src_g, dst_s, zeros)
    return list(res) if isinstance(res, (list, tuple)) else [res]


def _sc_propagate_edge(u, src_gw, dst_sw, zeros, w):
    @functools.partial(
        pl.kernel,
        out_type=[jax.ShapeDtypeStruct((NACC, w), jnp.float32)] * 2,
        mesh=_sc_mesh(),
        compiler_params=pltpu.CompilerParams(use_tc_tiling_on_sc=False),
        scratch_types=[
            pltpu.VMEM((CHW, CHUNK), jnp.int32),
            pltpu.VMEM((CHW, CHUNK), jnp.int32),
            pltpu.VMEM((KINF, CHUNK, w), jnp.float32),
            pltpu.VMEM_SHARED((N, w), jnp.float32),
            pltpu.VMEM_SHARED((NACC, w), jnp.float32),
            pltpu.SemaphoreType.DMA,
            pltpu.SemaphoreType.DMA,
        ],
    )
    def k(u_hbm, srcg_hbm, dsts_hbm, zeros_hbm, out0, out1,
          src_v, dst_v, rows_v, u_sp, acc, sem_g, sem_s):
        c = lax.axis_index("c")
        s = lax.axis_index("s")
        widx = c * NS + s

        @pl.when(s == 0)
        def _():
            pltpu.sync_copy(u_hbm, u_sp)

        pltpu.sync_copy(srcg_hbm.at[widx], src_v)
        pltpu.sync_copy(dsts_hbm.at[widx], dst_v)
        pltpu.sync_copy(zeros_hbm, acc.at[pl.ds(s * STRIPE, STRIPE)])
        plsc.subcore_barrier()

        def block(j, carry):
            base = j * KINF
            gets = [pltpu.async_copy(u_sp.at[src_v.at[base + t]],
                                     rows_v.at[t], sem_g)
                    for t in range(KINF)]
            puts = []
            for t in range(KINF):
                gets[t].wait()
                puts.append(pltpu.async_copy(rows_v.at[t],
                                             acc.at[dst_v.at[base + t]],
                                             sem_s, add=True))
            for d in puts:
                d.wait()
            return carry

        lax.fori_loop(0, CHW // KINF, block, 0)
        plsc.subcore_barrier()

        @pl.when(c == 0)
        def _():
            pltpu.sync_copy(acc.at[pl.ds(s * STRIPE, STRIPE)],
                            out0.at[pl.ds(s * STRIPE, STRIPE)])

        @pl.when(c == 1)
        def _():
            pltpu.sync_copy(acc.at[pl.ds(s * STRIPE, STRIPE)],
                            out1.at[pl.ds(s * STRIPE, STRIPE)])

    return list(k(u, src_gw, dst_sw, zeros))


def _sc_propagate(u_parts, edges, zeros, h):
    (src_g, dst_s), (src_gw, dst_sw) = edges
    outs, i = [], 0
    for call in CALLS[h]:
        if call[0] == "col":
            _, w, n = call
            outs += _sc_propagate_call(u_parts[i:i + n], src_g, dst_s,
                                       zeros[w], w, n)
            i += n
        else:
            w = call[1]
            outs += _sc_propagate_edge(u_parts[i], src_gw, dst_sw,
                                       zeros[w], w)
            i += 1
    return outs


def _sc_degree(src_s, ones, zeros):
    @functools.partial(
        pl.kernel,
        out_type=jax.ShapeDtypeStruct((NC, NACC, 16), jnp.float32),
        mesh=_sc_mesh(),
        compiler_params=pltpu.CompilerParams(use_tc_tiling_on_sc=False),
        scratch_types=[
            pltpu.VMEM((CHW, CHUNK), jnp.int32),
            pltpu.VMEM((CHUNK, 16), jnp.float32),
            pltpu.VMEM_SHARED((NACC, 16), jnp.float32),
            pltpu.SemaphoreType.DMA,
        ],
    )
    def k(srcs_hbm, ones_hbm, zeros_hbm, out_hbm, src_v, ones_v, acc, sem_s):
        c = lax.axis_index("c")
        s = lax.axis_index("s")
        widx = c * NS + s
        pltpu.sync_copy(srcs_hbm.at[widx], src_v)
        pltpu.sync_copy(ones_hbm, ones_v)
        pltpu.sync_copy(zeros_hbm, acc.at[pl.ds(s * STRIPE, STRIPE)])
        plsc.subcore_barrier()

        def block(j, carry):
            base = j * KINF
            puts = [pltpu.async_copy(ones_v, acc.at[src_v.at[base + t]],
                                     sem_s, add=True)
                    for t in range(KINF)]
            for d in puts:
                d.wait()
            return carry

        lax.fori_loop(0, CHW // KINF, block, 0)
        plsc.subcore_barrier()
        pltpu.sync_copy(acc.at[pl.ds(s * STRIPE, STRIPE)],
                        out_hbm.at[c, pl.ds(s * STRIPE, STRIPE)])

    return k(src_s, ones, zeros)


def _dis_tc(deg_p):
    def body(p0_ref, p1_ref, o_ref):
        d = p0_ref[0] + p1_ref[0]
        o_ref[...] = jnp.where(d > 0, lax.rsqrt(d), 0.0)

    return pl.pallas_call(
        body,
        grid=(GRID,),
        in_specs=[pl.BlockSpec((1, RB, 16), lambda i: (0, i, 0)),
                  pl.BlockSpec((1, RB, 16), lambda i: (1, i, 0))],
        out_specs=pl.BlockSpec((RB, 16), lambda i: (i, 0)),
        out_shape=jax.ShapeDtypeStruct((N, 16), jnp.float32),
    )(deg_p, deg_p)


def _col_offsets(h):
    offs, o = [], 0
    for w in PARTS[h]:
        offs.append(o)
        o += w
    return offs


def _prescale(x, dis, h):
    parts = PARTS[h]
    offs = _col_offsets(h)

    def body(x_ref, d_ref, *o_refs):
        u = x_ref[...] * d_ref[:, 0:1]
        for r, w, o in zip(o_refs, parts, offs):
            r[...] = u[:, o:o + w]

    return pl.pallas_call(
        body,
        grid=(GRID,),
        in_specs=[pl.BlockSpec((RB, h), lambda i: (i, 0)),
                  pl.BlockSpec((RB, 16), lambda i: (i, 0))],
        out_specs=[pl.BlockSpec((RB, w), lambda i: (i, 0)) for w in parts],
        out_shape=[jax.ShapeDtypeStruct((N, w), jnp.float32) for w in parts],
    )(x, dis)


def _combine(p_parts, dis, a, terms, h, relu=False, bias=None, want_u=True):
    parts = PARTS[h]
    offs = _col_offsets(h)
    np_ = len(p_parts)
    g = np_ // len(parts)
    nt = len(terms)
    nb = 1 if bias is not None else 0

    def body(*refs):
        ps = [sum(refs[i * g + j][...] for j in range(g))
              for i in range(len(parts))]
        psum = ps[0] if len(parts) == 1 else jnp.concatenate(ps, axis=1)
        dcol = refs[np_][:, 0:1]
        t = a * dcol * psum
        for (_, sgn), r in zip(terms, refs[np_ + 1:np_ + 1 + nt]):
            t = t + sgn * r[...]
        if bias is not None:
            t = t + refs[np_ + 1 + nt][...]
        if relu:
            t = jnp.maximum(t, 0.0)
        out0 = np_ + 1 + nt + nb
        refs[out0][...] = t
        if want_u:
            u = dcol * t
            for i, (w, o) in enumerate(zip(parts, offs)):
                refs[out0 + 1 + i][...] = u[:, o:o + w]

    in_specs, args = [], []
    pws = [w for w in parts for _ in range(g)]
    for p, w in zip(p_parts, pws):
        in_specs.append(pl.BlockSpec((RB, w), lambda i: (i, 0)))
        args.append(p)
    in_specs.append(pl.BlockSpec((RB, 16), lambda i: (i, 0)))
    args.append(dis)
    for (arr, _) in terms:
        in_specs.append(pl.BlockSpec((RB, h), lambda i: (i, 0)))
        args.append(arr)
    if bias is not None:
        in_specs.append(pl.BlockSpec((1, h), lambda i: (0, 0)))
        args.append(bias)
    out_shape = [jax.ShapeDtypeStruct((N, h), jnp.float32)]
    out_specs = [pl.BlockSpec((RB, h), lambda i: (i, 0))]
    if want_u:
        for w in parts:
            out_shape.append(jax.ShapeDtypeStruct((N, w), jnp.float32))
            out_specs.append(pl.BlockSpec((RB, w), lambda i: (i, 0)))
    res = pl.pallas_call(
        body, grid=(GRID,), in_specs=in_specs,
        out_specs=out_specs, out_shape=out_shape,
    )(*args)
    if want_u:
        return res[0], list(res[1:])
    return res[0]


def _mm_cheb4(ts, w, b):
    f, c = w.shape[1], w.shape[2]

    def body(t0, t1, t2, t3, w_ref, b_ref, o_ref):
        acc = jnp.dot(t0[...], w_ref[0], preferred_element_type=jnp.float32)
        acc = acc + jnp.dot(t1[...], w_ref[1], preferred_element_type=jnp.float32)
        acc = acc + jnp.dot(t2[...], w_ref[2], preferred_element_type=jnp.float32)
        acc = acc + jnp.dot(t3[...], w_ref[3], preferred_element_type=jnp.float32)
        o_ref[...] = jnp.maximum(acc + b_ref[...], 0.0)

    return pl.pallas_call(
        body,
        grid=(GRID,),
        in_specs=[pl.BlockSpec((RB, f), lambda i: (i, 0)),
                  pl.BlockSpec((RB, f), lambda i: (i, 0)),
                  pl.BlockSpec((RB, f), lambda i: (i, 0)),
                  pl.BlockSpec((RB, f), lambda i: (i, 0)),
                  pl.BlockSpec((4, f, c), lambda i: (0, 0, 0)),
                  pl.BlockSpec((1, c), lambda i: (0, 0))],
        out_specs=pl.BlockSpec((RB, c), lambda i: (i, 0)),
        out_shape=jax.ShapeDtypeStruct((N, c), jnp.float32),
    )(ts[0], ts[1], ts[2], ts[3], w, b)


def _mm_split(hmat, wp):
    f, c = wp.shape
    h = c // 4

    def body(h_ref, w_ref, o0, o1, o2, o3):
        acc = jnp.dot(h_ref[...], w_ref[...],
                      preferred_element_type=jnp.float32)
        o0[...] = acc[:, 0 * h:1 * h]
        o1[...] = acc[:, 1 * h:2 * h]
        o2[...] = acc[:, 2 * h:3 * h]
        o3[...] = acc[:, 3 * h:4 * h]

    return pl.pallas_call(
        body,
        grid=(GRID,),
        in_specs=[pl.BlockSpec((RB, f), lambda i: (i, 0)),
                  pl.BlockSpec((f, c), lambda i: (0, 0))],
        out_specs=[pl.BlockSpec((RB, h), lambda i: (i, 0))] * 4,
        out_shape=[jax.ShapeDtypeStruct((N, h), jnp.float32)] * 4,
    )(hmat, wp)


def _layer1(x, dis, edges, zeros, w1, b1):
    u0 = _prescale(x, dis, 128)
    p1 = _sc_propagate(u0, edges, zeros, 128)
    t1, u1 = _combine(p1, dis, -1.0, [], 128)
    p2 = _sc_propagate(u1, edges, zeros, 128)
    t2, u2 = _combine(p2, dis, -2.0, [(x, -1.0)], 128)
    p3 = _sc_propagate(u2, edges, zeros, 128)
    t3 = _combine(p3, dis, -2.0, [(t1, -1.0)], 128, want_u=False)
    return _mm_cheb4([x, t1, t2, t3], w1, b1)


def _layer_clenshaw(zs, dis, edges, zeros, h, bias):
    u3 = _prescale(zs[3], dis, h)
    p = _sc_propagate(u3, edges, zeros, h)
    c2, u2 = _combine(p, dis, -2.0, [(zs[2], 1.0)], h)
    p = _sc_propagate(u2, edges, zeros, h)
    c1, u1 = _combine(p, dis, -2.0, [(zs[1], 1.0), (zs[3], -1.0)], h)
    p = _sc_propagate(u1, edges, zeros, h)
    out = _combine(p, dis, -1.0, [(zs[0], 1.0), (c2, -1.0)], h,
                   relu=True, bias=bias, want_u=False)
    return out


def kernel(x, edge_index, W1, b1, W2, b2, W3, b3):
    ei = edge_index.astype(jnp.int32)
    src, dst = ei[0], ei[1]
    pad = EPAD - E
    src_g = jnp.pad(src, (0, pad)).reshape(NS, CH2, CHUNK)
    dst_s = jnp.pad(dst, (0, pad), constant_values=N).reshape(NS, CH2, CHUNK)
    src_gw = jnp.pad(src, (0, pad)).reshape(NW, CHW, CHUNK)
    dst_sw = jnp.pad(dst, (0, pad), constant_values=N).reshape(NW, CHW, CHUNK)
    src_s = jnp.pad(src, (0, pad), constant_values=N).reshape(NW, CHW, CHUNK)
    edges = ((src_g, dst_s), (src_gw, dst_sw))

    ones16 = jnp.ones((CHUNK, 16), jnp.float32)
    zeros = {w: jnp.zeros((STRIPE, w), jnp.float32) for w in (64, 48, 16)}

    w2p = jnp.pad(W2, ((0, 0), (0, 0), (0, 24))).transpose(1, 0, 2).reshape(400, 4 * 224)
    w3p = jnp.pad(W3, ((0, 0), (0, 24), (0, 12))).transpose(1, 0, 2).reshape(224, 4 * 16)
    b1r = b1.reshape(1, 400)
    b2p = jnp.pad(b2, (0, 24)).reshape(1, 224)
    b3p = jnp.pad(b3, (0, 12)).reshape(1, 16)

    deg_p = _sc_degree(src_s, ones16, zeros[16])
    dis = _dis_tc(deg_p)

    h1 = _layer1(x, dis, edges, zeros, W1, b1r)
    z2 = _mm_split(h1, w2p)
    h2 = _layer_clenshaw(z2, dis, edges, zeros, 224, b2p)
    z3 = _mm_split(h2, w3p)
    h3 = _layer_clenshaw(z3, dis, edges, zeros, 16, b3p)
    return h3[:, :4]
